# baseline (speedup 1.0000x reference)
import functools

import jax
import jax.numpy as jnp
from jax import lax
from jax.experimental import pallas as pl
from jax.experimental.pallas import tpu as pltpu
from jax.experimental.pallas import tpu_sc as plsc

IN_F = 4096
OUT_F = 4096
W_ELEMS = OUT_F * IN_F

NUM_CORES = 2
NUM_SUBCORES = 16
WIN = 2 ** 20
SLICE = WIN // NUM_SUBCORES
ZCHUNK = 8192
DMA_E = 128


def _densify(coords_p, weights_p, base_win, num_win):
    n = coords_p.shape[0]
    chunk = n // NUM_SUBCORES
    ndma = chunk // DMA_E
    passes = num_win // NUM_CORES

    mesh = plsc.VectorSubcoreMesh(core_axis_name="c", subcore_axis_name="s")

    @functools.partial(
        pl.kernel,
        out_type=jax.ShapeDtypeStruct((num_win * WIN,), jnp.float32),
        mesh=mesh,
        scratch_types=[
            pltpu.VMEM((chunk,), jnp.int32),
            pltpu.VMEM((chunk,), jnp.float32),
            pltpu.VMEM((ndma, DMA_E), jnp.int32),
            pltpu.VMEM((chunk,), jnp.float32),
            pltpu.VMEM((ZCHUNK,), jnp.float32),
            pltpu.VMEM_SHARED((WIN,), jnp.float32),
            pltpu.SemaphoreType.DMA,
        ],
    )
    def densify_kernel(coords_hbm, weights_hbm, w_hbm,
                       c_v, w_v, idx_v, val_v, z_v, win_sh, sem):
        c = lax.axis_index("c")
        s = lax.axis_index("s")
        chunk_off = s * chunk
        slice_off = s * SLICE
        pltpu.sync_copy(coords_hbm.at[pl.ds(chunk_off, chunk)], c_v)
        pltpu.sync_copy(weights_hbm.at[pl.ds(chunk_off, chunk)], w_v)

        def zbody(i, carry):
            z_v[pl.ds(i * 16, 16)] = jnp.zeros((16,), jnp.float32)
            return carry
        lax.fori_loop(0, ZCHUNK // 16, zbody, 0)

        lanes = lax.broadcasted_iota(jnp.int32, (16,), 0)

        for p in range(passes):
            local_win = p * NUM_CORES + c
            ebase = (base_win + local_win) * WIN
            obase = local_win * WIN

            for q in range(SLICE // ZCHUNK):
                pltpu.sync_copy(z_v, win_sh.at[pl.ds(slice_off + q * ZCHUNK, ZCHUNK)])

            def fbody(i, carry):
                cs = c_v[pl.ds(i * 16, 16)]
                ws = w_v[pl.ds(i * 16, 16)]
                loc = cs - ebase
                m = (loc >= 0) & (loc < WIN)
                dump = chunk_off + i * 16 + lanes
                idx_v[i // 8, pl.ds((i % 8) * 16, 16)] = jnp.where(m, loc, dump)
                val_v[pl.ds(i * 16, 16)] = jnp.where(m, ws, jnp.float32(0.0))
                return carry
            lax.fori_loop(0, chunk // 16, fbody, 0)

            plsc.subcore_barrier()

            def dbody(j, carry):
                pltpu.async_copy(val_v.at[pl.ds(j * DMA_E, DMA_E)],
                                 win_sh.at[idx_v.at[j]], sem, add=True)
                return carry
            lax.fori_loop(0, ndma, dbody, 0)
            pltpu.make_async_copy(weights_hbm.at[pl.ds(0, chunk)], val_v, sem).wait()

            plsc.subcore_barrier()

            pltpu.sync_copy(win_sh.at[pl.ds(slice_off, SLICE)],
                            w_hbm.at[pl.ds(obase + slice_off, SLICE)])

    return densify_kernel(coords_p, weights_p)


def _retile_body(f_ref, o_ref):
    o_ref[...] = f_ref[...].reshape(o_ref.shape).astype(jnp.bfloat16)


def _retile(w_flat, rows, br=128):
    return pl.pallas_call(
        _retile_body,
        grid=(rows // br,),
        in_specs=[pl.BlockSpec((br * IN_F,), lambda i: (i,))],
        out_specs=pl.BlockSpec((br, IN_F), lambda i: (i, 0)),
        out_shape=jax.ShapeDtypeStruct((rows, IN_F), jnp.bfloat16),
    )(w_flat)


def _mm_first_body(x_ref, w_ref, o_ref):
    xb = x_ref[...].astype(jnp.bfloat16)
    o_ref[...] = jax.lax.dot_general(
        xb, w_ref[...], (((1,), (1,)), ((), ())),
        preferred_element_type=jnp.float32)


def _mm_rest_body(prev_ref, x_ref, w_ref, o_ref):
    del prev_ref
    xb = x_ref[...].astype(jnp.bfloat16)
    o_ref[...] = jax.lax.dot_general(
        xb, w_ref[...], (((1,), (1,)), ((), ())),
        preferred_element_type=jnp.float32)


def _matmul_half(x2d, w_half, col_block, prev_out=None, bm=256):
    m = x2d.shape[0]
    bn = w_half.shape[0]
    grid = (m // bm,)
    x_spec = pl.BlockSpec((bm, IN_F), lambda i: (i, 0))
    w_spec = pl.BlockSpec((bn, IN_F), lambda i: (0, 0))
    out_spec = pl.BlockSpec((bm, bn), lambda i, _c=col_block: (i, _c))
    out_shape = jax.ShapeDtypeStruct((m, OUT_F), jnp.float32)
    if prev_out is None:
        return pl.pallas_call(
            _mm_first_body,
            grid=grid,
            in_specs=[x_spec, w_spec],
            out_specs=out_spec,
            out_shape=out_shape,
        )(x2d, w_half)
    return pl.pallas_call(
        _mm_rest_body,
        grid=grid,
        in_specs=[pl.BlockSpec(memory_space=pl.ANY), x_spec, w_spec],
        out_specs=out_spec,
        out_shape=out_shape,
        input_output_aliases={0: 0},
    )(prev_out, x2d, w_half)


def kernel(x, coords, weights):
    coords = coords.astype(jnp.int32)
    w32 = weights.astype(jnp.float32)
    nnz = coords.shape[0]
    unit = NUM_SUBCORES * DMA_E
    n_pad = ((nnz + unit - 1) // unit) * unit
    coords_p = jnp.pad(coords, (0, n_pad - nnz))
    w_p = jnp.pad(w32, (0, n_pad - nnz))

    num_win = W_ELEMS // WIN
    halves = 4
    win_per_half = num_win // halves
    rows_per_half = OUT_F // halves

    orig_shape = x.shape
    x2d = x.reshape(-1, IN_F)

    out = None
    for h in range(halves):
        w_flat_h = _densify(coords_p, w_p, h * win_per_half, win_per_half)
        w_bf16_h = _retile(w_flat_h, rows_per_half)
        out = _matmul_half(x2d, w_bf16_h, h, prev_out=out)

    return out.reshape(orig_shape[:-1] + (OUT_F,)).astype(x.dtype)


# speedup vs baseline: 1.0597x; 1.0597x over previous
import functools

import jax
import jax.numpy as jnp
from jax import lax
from jax.experimental import pallas as pl
from jax.experimental.pallas import tpu as pltpu
from jax.experimental.pallas import tpu_sc as plsc

IN_F = 4096
OUT_F = 4096
W_ELEMS = OUT_F * IN_F

NUM_CORES = 2
NUM_SUBCORES = 16
WIN = 2 ** 20
SLICE = WIN // NUM_SUBCORES
ZCHUNK = 8192
DMA_E = 128


def _densify(coords_p, weights_p, base_win, num_win):
    n = coords_p.shape[0]
    chunk = n // NUM_SUBCORES
    ndma = chunk // DMA_E
    passes = num_win // NUM_CORES

    mesh = plsc.VectorSubcoreMesh(core_axis_name="c", subcore_axis_name="s")

    @functools.partial(
        pl.kernel,
        out_type=jax.ShapeDtypeStruct((num_win * WIN,), jnp.float32),
        mesh=mesh,
        scratch_types=[
            pltpu.VMEM((chunk,), jnp.int32),
            pltpu.VMEM((chunk,), jnp.float32),
            pltpu.VMEM((ndma, DMA_E), jnp.int32),
            pltpu.VMEM((chunk,), jnp.float32),
            pltpu.VMEM((ZCHUNK,), jnp.float32),
            pltpu.VMEM_SHARED((WIN,), jnp.float32),
            pltpu.SemaphoreType.DMA,
        ],
    )
    def densify_kernel(coords_hbm, weights_hbm, w_hbm,
                       c_v, w_v, idx_v, val_v, z_v, win_sh, sem):
        c = lax.axis_index("c")
        s = lax.axis_index("s")
        chunk_off = s * chunk
        slice_off = s * SLICE
        pltpu.sync_copy(coords_hbm.at[pl.ds(chunk_off, chunk)], c_v)
        pltpu.sync_copy(weights_hbm.at[pl.ds(chunk_off, chunk)], w_v)

        def zbody(i, carry):
            z_v[pl.ds(i * 16, 16)] = jnp.zeros((16,), jnp.float32)
            return carry
        lax.fori_loop(0, ZCHUNK // 16, zbody, 0)

        lanes = lax.broadcasted_iota(jnp.int32, (16,), 0)

        for p in range(passes):
            local_win = p * NUM_CORES + c
            ebase = (base_win + local_win) * WIN
            obase = local_win * WIN

            for q in range(SLICE // ZCHUNK):
                pltpu.sync_copy(z_v, win_sh.at[pl.ds(slice_off + q * ZCHUNK, ZCHUNK)])

            def fbody(i, carry):
                cs = c_v[pl.ds(i * 16, 16)]
                ws = w_v[pl.ds(i * 16, 16)]
                loc = cs - ebase
                m = (loc >= 0) & (loc < WIN)
                dump = chunk_off + i * 16 + lanes
                idx_v[i // 8, pl.ds((i % 8) * 16, 16)] = jnp.where(m, loc, dump)
                val_v[pl.ds(i * 16, 16)] = jnp.where(m, ws, jnp.float32(0.0))
                return carry
            lax.fori_loop(0, chunk // 16, fbody, 0)

            plsc.subcore_barrier()

            def dbody(j, carry):
                pltpu.async_copy(val_v.at[pl.ds(j * DMA_E, DMA_E)],
                                 win_sh.at[idx_v.at[j]], sem, add=True)
                return carry
            lax.fori_loop(0, ndma, dbody, 0)
            pltpu.make_async_copy(weights_hbm.at[pl.ds(0, chunk)], val_v, sem).wait()

            plsc.subcore_barrier()

            pltpu.sync_copy(win_sh.at[pl.ds(slice_off, SLICE)],
                            w_hbm.at[pl.ds(obase + slice_off, SLICE)])

    return densify_kernel(coords_p, weights_p)


def _retile_body(f_ref, o_ref):
    o_ref[...] = f_ref[...].reshape(o_ref.shape).astype(jnp.bfloat16)


def _retile(w_flat, rows, br=128):
    return pl.pallas_call(
        _retile_body,
        grid=(rows // br,),
        in_specs=[pl.BlockSpec((br * IN_F,), lambda i: (i,))],
        out_specs=pl.BlockSpec((br, IN_F), lambda i: (i, 0)),
        out_shape=jax.ShapeDtypeStruct((rows, IN_F), jnp.bfloat16),
    )(w_flat)


def _mm_first_body(x_ref, w_ref, o_ref):
    xb = x_ref[...].astype(jnp.bfloat16)
    o_ref[...] = jax.lax.dot_general(
        xb, w_ref[...], (((1,), (1,)), ((), ())),
        preferred_element_type=jnp.float32)


def _mm_rest_body(prev_ref, x_ref, w_ref, o_ref):
    del prev_ref
    xb = x_ref[...].astype(jnp.bfloat16)
    o_ref[...] = jax.lax.dot_general(
        xb, w_ref[...], (((1,), (1,)), ((), ())),
        preferred_element_type=jnp.float32)


def _matmul_half(x2d, w_half, col_block, prev_out=None, bm=512):
    m = x2d.shape[0]
    bn = w_half.shape[0]
    grid = (m // bm,)
    x_spec = pl.BlockSpec((bm, IN_F), lambda i: (i, 0))
    w_spec = pl.BlockSpec((bn, IN_F), lambda i: (0, 0))
    out_spec = pl.BlockSpec((bm, bn), lambda i, _c=col_block: (i, _c))
    out_shape = jax.ShapeDtypeStruct((m, OUT_F), jnp.float32)
    if prev_out is None:
        return pl.pallas_call(
            _mm_first_body,
            grid=grid,
            in_specs=[x_spec, w_spec],
            out_specs=out_spec,
            out_shape=out_shape,
        )(x2d, w_half)
    return pl.pallas_call(
        _mm_rest_body,
        grid=grid,
        in_specs=[pl.BlockSpec(memory_space=pl.ANY), x_spec, w_spec],
        out_specs=out_spec,
        out_shape=out_shape,
        input_output_aliases={0: 0},
    )(prev_out, x2d, w_half)


def kernel(x, coords, weights):
    coords = coords.astype(jnp.int32)
    w32 = weights.astype(jnp.float32)
    nnz = coords.shape[0]
    unit = NUM_SUBCORES * DMA_E
    n_pad = ((nnz + unit - 1) // unit) * unit
    coords_p = jnp.pad(coords, (0, n_pad - nnz))
    w_p = jnp.pad(w32, (0, n_pad - nnz))

    num_win = W_ELEMS // WIN
    halves = 2
    win_per_half = num_win // halves
    rows_per_half = OUT_F // halves

    orig_shape = x.shape
    x2d = x.reshape(-1, IN_F)

    out = None
    for h in range(halves):
        w_flat_h = _densify(coords_p, w_p, h * win_per_half, win_per_half)
        w_bf16_h = _retile(w_flat_h, rows_per_half)
        out = _matmul_half(x2d, w_bf16_h, h, prev_out=out)

    return out.reshape(orig_shape[:-1] + (OUT_F,)).astype(x.dtype)


# speedup vs baseline: 1.0929x; 1.0313x over previous
import functools

import jax
import jax.numpy as jnp
from jax import lax
from jax.experimental import pallas as pl
from jax.experimental.pallas import tpu as pltpu
from jax.experimental.pallas import tpu_sc as plsc

IN_F = 4096
OUT_F = 4096
W_ELEMS = OUT_F * IN_F

NUM_CORES = 2
NUM_SUBCORES = 16
WIN = 2 ** 20
SLICE = WIN // NUM_SUBCORES
ZCHUNK = 8192
DMA_E = 128


def _densify(coords_p, weights_p, base_win, num_win):
    n = coords_p.shape[0]
    chunk = n // NUM_SUBCORES
    ndma = chunk // DMA_E
    passes = num_win // NUM_CORES

    mesh = plsc.VectorSubcoreMesh(core_axis_name="c", subcore_axis_name="s")

    @functools.partial(
        pl.kernel,
        out_type=jax.ShapeDtypeStruct((num_win * WIN,), jnp.float32),
        mesh=mesh,
        scratch_types=[
            pltpu.VMEM((chunk,), jnp.int32),
            pltpu.VMEM((chunk,), jnp.float32),
            pltpu.VMEM((ndma, DMA_E), jnp.int32),
            pltpu.VMEM((chunk,), jnp.float32),
            pltpu.VMEM((ZCHUNK,), jnp.float32),
            pltpu.VMEM_SHARED((WIN,), jnp.float32),
            pltpu.SemaphoreType.DMA,
        ],
    )
    def densify_kernel(coords_hbm, weights_hbm, w_hbm,
                       c_v, w_v, idx_v, val_v, z_v, win_sh, sem):
        c = lax.axis_index("c")
        s = lax.axis_index("s")
        chunk_off = s * chunk
        slice_off = s * SLICE
        pltpu.sync_copy(coords_hbm.at[pl.ds(chunk_off, chunk)], c_v)
        pltpu.sync_copy(weights_hbm.at[pl.ds(chunk_off, chunk)], w_v)

        def zbody(i, carry):
            z_v[pl.ds(i * 16, 16)] = jnp.zeros((16,), jnp.float32)
            return carry
        lax.fori_loop(0, ZCHUNK // 16, zbody, 0)

        lanes = lax.broadcasted_iota(jnp.int32, (16,), 0)

        for p in range(passes):
            local_win = p * NUM_CORES + c
            ebase = (base_win + local_win) * WIN
            obase = local_win * WIN

            for q in range(SLICE // ZCHUNK):
                pltpu.sync_copy(z_v, win_sh.at[pl.ds(slice_off + q * ZCHUNK, ZCHUNK)])

            def fbody(i, carry):
                cs = c_v[pl.ds(i * 16, 16)]
                ws = w_v[pl.ds(i * 16, 16)]
                loc = cs - ebase
                m = (loc >= 0) & (loc < WIN)
                dump = chunk_off + i * 16 + lanes
                idx_v[i // 8, pl.ds((i % 8) * 16, 16)] = jnp.where(m, loc, dump)
                val_v[pl.ds(i * 16, 16)] = jnp.where(m, ws, jnp.float32(0.0))
                return carry
            lax.fori_loop(0, chunk // 16, fbody, 0)

            plsc.subcore_barrier()

            def dbody(j, carry):
                pltpu.async_copy(val_v.at[pl.ds(j * DMA_E, DMA_E)],
                                 win_sh.at[idx_v.at[j]], sem, add=True)
                return carry
            lax.fori_loop(0, ndma, dbody, 0)
            pltpu.make_async_copy(weights_hbm.at[pl.ds(0, chunk)], val_v, sem).wait()

            plsc.subcore_barrier()

            pltpu.sync_copy(win_sh.at[pl.ds(slice_off, SLICE)],
                            w_hbm.at[pl.ds(obase + slice_off, SLICE)])

    return densify_kernel(coords_p, weights_p)


def _retile_body(f_ref, o_ref):
    o_ref[...] = f_ref[...].reshape(o_ref.shape).astype(jnp.bfloat16)


def _retile(w_flat, rows, br=128):
    return pl.pallas_call(
        _retile_body,
        grid=(rows // br,),
        in_specs=[pl.BlockSpec((br * IN_F,), lambda i: (i,))],
        out_specs=pl.BlockSpec((br, IN_F), lambda i: (i, 0)),
        out_shape=jax.ShapeDtypeStruct((rows, IN_F), jnp.bfloat16),
    )(w_flat)


def _mm_first_body(x_ref, w_ref, o_ref):
    xb = x_ref[...].astype(jnp.bfloat16)
    o_ref[...] = jax.lax.dot_general(
        xb, w_ref[...], (((1,), (1,)), ((), ())),
        preferred_element_type=jnp.float32)


def _mm_rest_body(prev_ref, x_ref, w_ref, o_ref):
    del prev_ref
    xb = x_ref[...].astype(jnp.bfloat16)
    o_ref[...] = jax.lax.dot_general(
        xb, w_ref[...], (((1,), (1,)), ((), ())),
        preferred_element_type=jnp.float32)


def _matmul_half(x2d, w_half, col_block, prev_out=None, bm=512):
    m = x2d.shape[0]
    bn = w_half.shape[0]
    grid = (m // bm,)
    x_spec = pl.BlockSpec((bm, IN_F), lambda i: (i, 0))
    w_spec = pl.BlockSpec((bn, IN_F), lambda i: (0, 0))
    out_spec = pl.BlockSpec((bm, bn), lambda i, _c=col_block: (i, _c))
    out_shape = jax.ShapeDtypeStruct((m, OUT_F), jnp.float32)
    if prev_out is None:
        return pl.pallas_call(
            _mm_first_body,
            grid=grid,
            in_specs=[x_spec, w_spec],
            out_specs=out_spec,
            out_shape=out_shape,
        )(x2d, w_half)
    return pl.pallas_call(
        _mm_rest_body,
        grid=grid,
        in_specs=[pl.BlockSpec(memory_space=pl.ANY), x_spec, w_spec],
        out_specs=out_spec,
        out_shape=out_shape,
        input_output_aliases={0: 0},
    )(prev_out, x2d, w_half)


def kernel(x, coords, weights):
    coords = coords.astype(jnp.int32)
    w32 = weights.astype(jnp.float32)
    nnz = coords.shape[0]
    unit = NUM_SUBCORES * DMA_E
    n_pad = ((nnz + unit - 1) // unit) * unit
    coords_p = jnp.pad(coords, (0, n_pad - nnz))
    w_p = jnp.pad(w32, (0, n_pad - nnz))

    rows_per_win = WIN // IN_F
    segments = [(0, 4), (4, 4), (8, 8)]

    orig_shape = x.shape
    x2d = x.reshape(-1, IN_F)

    out = None
    for w0, nw in segments:
        w_flat_h = _densify(coords_p, w_p, w0, nw)
        rows = nw * rows_per_win
        w_bf16_h = _retile(w_flat_h, rows)
        out = _matmul_half(x2d, w_bf16_h, (w0 * rows_per_win) // rows,
                           prev_out=out)

    return out.reshape(orig_shape[:-1] + (OUT_F,)).astype(x.dtype)
